# baseline (speedup 1.0000x reference)
import jax
import jax.numpy as jnp
from jax.experimental import pallas as pl
from jax.experimental.pallas import tpu as pltpu


def _value_sa_body(x_ref, u_ref, w1x_ref, w1u_ref, b1_ref,
                   w2_ref, b2_ref, w3_ref, b3_ref, o_ref):
    xb = x_ref[...].astype(jnp.bfloat16)
    ub = u_ref[...].astype(jnp.bfloat16)
    h1 = jnp.dot(xb, w1x_ref[...].astype(jnp.bfloat16),
                 preferred_element_type=jnp.float32)
    h1 = h1 + jnp.dot(ub, w1u_ref[...].astype(jnp.bfloat16),
                      preferred_element_type=jnp.float32)
    h1 = jnp.maximum(h1 + b1_ref[...], 0.0)

    h2 = jnp.dot(h1.astype(jnp.bfloat16), w2_ref[...].astype(jnp.bfloat16),
                 preferred_element_type=jnp.float32)
    h2 = jnp.maximum(h2 + b2_ref[...], 0.0)

    res = jnp.sum(h2 * w3_ref[...], axis=-1)
    res = res.reshape(res.shape[0] // 128, 128)
    o_ref[...] = (res + b3_ref[0, 0]).astype(o_ref.dtype)[None]


def kernel(x, u, w1x, w1u, b1, w2, b2, w3, b3, *, batch_tile=4096):
    B, obs_dim = x.shape
    _, action_dim = u.shape
    H = w1x.shape[1]

    TB = batch_tile
    while B % TB != 0:
        TB //= 2
    nb = B // TB

    inv = lambda shape: pl.BlockSpec(shape, lambda i: (0, 0))
    tiled = lambda shape: pl.BlockSpec(shape, lambda i: (i, 0))

    out = pl.pallas_call(
        _value_sa_body,
        out_shape=jax.ShapeDtypeStruct((nb, TB // 128, 128), x.dtype),
        grid=(nb,),
        in_specs=[
            tiled((TB, obs_dim)),
            tiled((TB, action_dim)),
            inv((obs_dim, H)),
            inv((action_dim, H)),
            inv((1, H)),
            inv((H, H)),
            inv((1, H)),
            inv((1, H)),
            pl.BlockSpec(memory_space=pltpu.MemorySpace.SMEM),
        ],
        out_specs=pl.BlockSpec((1, TB // 128, 128), lambda i: (i, 0, 0)),
        compiler_params=pltpu.CompilerParams(
            dimension_semantics=("parallel",),
        ),
    )(x, u, w1x, w1u, b1, w2, b2, w3, b3)
    return out.reshape(B, 1)


# speedup vs baseline: 1.1390x; 1.1390x over previous
import jax
import jax.numpy as jnp
from jax.experimental import pallas as pl
from jax.experimental.pallas import tpu as pltpu


def _value_sa_body(x_ref, u_ref, w1x_ref, w1u_ref, b1_ref,
                   w2_ref, b2_ref, w3_ref, b3_ref, o_ref):
    xb = x_ref[...].astype(jnp.bfloat16)
    ub = u_ref[...].astype(jnp.bfloat16)
    h1 = jnp.dot(xb, w1x_ref[...].astype(jnp.bfloat16),
                 preferred_element_type=jnp.float32)
    h1 = h1 + jnp.dot(ub, w1u_ref[...].astype(jnp.bfloat16),
                      preferred_element_type=jnp.float32)
    h1 = jnp.maximum(h1 + b1_ref[...], 0.0)

    h2 = jnp.dot(h1.astype(jnp.bfloat16), w2_ref[...].astype(jnp.bfloat16),
                 preferred_element_type=jnp.float32)
    h2 = jnp.maximum(h2 + b2_ref[...], 0.0)

    res = jnp.sum(h2 * w3_ref[...], axis=-1)
    res = res.reshape(res.shape[0] // 128, 128)
    o_ref[...] = (res + b3_ref[0, 0]).astype(o_ref.dtype)[None]


def kernel(x, u, w1x, w1u, b1, w2, b2, w3, b3, *, batch_tile=8192):
    B, obs_dim = x.shape
    _, action_dim = u.shape
    H = w1x.shape[1]

    TB = batch_tile
    while B % TB != 0:
        TB //= 2
    nb = B // TB

    inv = lambda shape: pl.BlockSpec(shape, lambda i: (0, 0))
    tiled = lambda shape: pl.BlockSpec(shape, lambda i: (i, 0))

    out = pl.pallas_call(
        _value_sa_body,
        out_shape=jax.ShapeDtypeStruct((nb, TB // 128, 128), x.dtype),
        grid=(nb,),
        in_specs=[
            tiled((TB, obs_dim)),
            tiled((TB, action_dim)),
            inv((obs_dim, H)),
            inv((action_dim, H)),
            inv((1, H)),
            inv((H, H)),
            inv((1, H)),
            inv((1, H)),
            pl.BlockSpec(memory_space=pltpu.MemorySpace.SMEM),
        ],
        out_specs=pl.BlockSpec((1, TB // 128, 128), lambda i: (i, 0, 0)),
        compiler_params=pltpu.CompilerParams(
            dimension_semantics=("parallel",),
        ),
    )(x, u, w1x, w1u, b1, w2, b2, w3, b3)
    return out.reshape(B, 1)


# speedup vs baseline: 1.1448x; 1.0051x over previous
import jax
import jax.numpy as jnp
from jax.experimental import pallas as pl
from jax.experimental.pallas import tpu as pltpu


def _value_sa_body(x_ref, u_ref, w1x_ref, w1u_ref, b1_ref,
                   w2_ref, b2_ref, w3_ref, b3_ref, o_ref):
    xb = x_ref[...].astype(jnp.bfloat16)
    ub = u_ref[...].astype(jnp.bfloat16)
    h1 = jnp.dot(xb, w1x_ref[...].astype(jnp.bfloat16),
                 preferred_element_type=jnp.float32)
    h1 = h1 + jnp.dot(ub, w1u_ref[...].astype(jnp.bfloat16),
                      preferred_element_type=jnp.float32)
    h1b = jnp.maximum(h1.astype(jnp.bfloat16)
                      + b1_ref[...].astype(jnp.bfloat16),
                      jnp.bfloat16(0.0))

    h2 = jnp.dot(h1b, w2_ref[...].astype(jnp.bfloat16),
                 preferred_element_type=jnp.float32)
    h2 = jnp.maximum(h2 + b2_ref[...], 0.0)

    res = jnp.sum(h2 * w3_ref[...], axis=-1)
    res = res.reshape(res.shape[0] // 128, 128)
    o_ref[...] = (res + b3_ref[0, 0]).astype(o_ref.dtype)[None]


def kernel(x, u, w1x, w1u, b1, w2, b2, w3, b3, *, batch_tile=8192):
    B, obs_dim = x.shape
    _, action_dim = u.shape
    H = w1x.shape[1]

    TB = batch_tile
    while B % TB != 0:
        TB //= 2
    nb = B // TB

    inv = lambda shape: pl.BlockSpec(shape, lambda i: (0, 0))
    tiled = lambda shape: pl.BlockSpec(shape, lambda i: (i, 0))

    out = pl.pallas_call(
        _value_sa_body,
        out_shape=jax.ShapeDtypeStruct((nb, TB // 128, 128), x.dtype),
        grid=(nb,),
        in_specs=[
            tiled((TB, obs_dim)),
            tiled((TB, action_dim)),
            inv((obs_dim, H)),
            inv((action_dim, H)),
            inv((1, H)),
            inv((H, H)),
            inv((1, H)),
            inv((1, H)),
            pl.BlockSpec(memory_space=pltpu.MemorySpace.SMEM),
        ],
        out_specs=pl.BlockSpec((1, TB // 128, 128), lambda i: (i, 0, 0)),
        compiler_params=pltpu.CompilerParams(
            dimension_semantics=("parallel",),
        ),
    )(x, u, w1x, w1u, b1, w2, b2, w3, b3)
    return out.reshape(B, 1)
